# Initial kernel scaffold; baseline (speedup 1.0000x reference)
import functools

import jax
import jax.numpy as jnp
from jax import lax
from jax.experimental import pallas as pl
from jax.experimental.pallas import tpu as pltpu
from jax.experimental.pallas import tpu_sc as plsc

EMBED = 128
NC, NS = 2, 16
NW = NC * NS
K = 512


def _table_body(res_ref, let_ref, row_ref, col_ref, out_ref):
    i = lax.broadcasted_iota(jnp.int32, (480, EMBED), 0)
    s0 = i // 120
    s1 = (i // 30) % 4
    r = (i // 5) % 6
    c = i % 5

    def pick(ref, sel, n):
        acc = jnp.broadcast_to(ref[n - 1, :][None, :], (480, EMBED))
        for k in range(n - 2, -1, -1):
            row = jnp.broadcast_to(ref[k, :][None, :], (480, EMBED))
            acc = jnp.where(sel == k, row, acc)
        return acc

    out_ref[...] = (pick(res_ref, s0, 4) + pick(let_ref, s1, 4)
                    + pick(row_ref, r, 6) + pick(col_ref, c, 5))


def _build_table(result_table, letter_table, row_table, col_table):
    return pl.pallas_call(
        _table_body,
        out_shape=jax.ShapeDtypeStruct((480, EMBED), jnp.float32),
    )(result_table, letter_table, row_table, col_table)


@functools.lru_cache(maxsize=None)
def _make_gather(n_rows):
    assert n_rows % (NW * K) == 0
    rpw = n_rows // NW
    chunks = rpw // K
    mesh = plsc.VectorSubcoreMesh(core_axis_name="c", subcore_axis_name="s",
                                  num_cores=NC, num_subcores=NS)

    @functools.partial(
        pl.kernel,
        out_type=jax.ShapeDtypeStruct((n_rows, EMBED), jnp.float32),
        mesh=mesh,
        scratch_types=[
            pltpu.VMEM((2 * K,), jnp.int32),
            pltpu.VMEM((K // 128, 128), jnp.int32),
            pltpu.VMEM((K, EMBED), jnp.float32),
            pltpu.SemaphoreType.DMA,
        ],
    )
    def gather(t_hbm, st_hbm, out_hbm, state_v, idx_v, rows_v, sem):
        wid = lax.axis_index("s") * NC + lax.axis_index("c")
        w_base = wid * rpw
        lane = lax.iota(jnp.int32, 16)

        def chunk_body(g, carry):
            base = w_base + g * K
            pltpu.sync_copy(st_hbm.at[pl.ds(2 * base, 2 * K)], state_v)
            for j in range(K // 16):
                ii = base + j * 16 + lane
                s0 = plsc.load_gather(state_v, [j * 32 + 2 * lane])
                s1 = plsc.load_gather(state_v, [j * 32 + 2 * lane + 1])
                fused = (s0 * 4 + s1) * 30 + lax.rem(ii, 30)
                idx_v[j // 8, pl.ds((j % 8) * 16, 16)] = fused
            copies = [
                pltpu.async_copy(t_hbm.at[idx_v.at[j]],
                                 rows_v.at[pl.ds(j * 128, 128)], sem)
                for j in range(K // 128)
            ]
            for cp in copies:
                cp.wait()
            pltpu.sync_copy(rows_v, out_hbm.at[pl.ds(base, K)])
            return carry

        lax.fori_loop(0, chunks, chunk_body, 0)

    return gather


def kernel(state, result_table, letter_table, col_table, row_table):
    bn = state.shape[0]
    table = _build_table(result_table, letter_table, row_table, col_table)
    st_flat = state.reshape(-1)
    out = _make_gather(bn * 30)(table, st_flat)
    return out.reshape(bn, 6, 5, EMBED)


# speedup vs baseline: 8.6337x; 8.6337x over previous
import functools

import jax
import jax.numpy as jnp
from jax import lax
from jax.experimental import pallas as pl
from jax.experimental.pallas import tpu as pltpu
from jax.experimental.pallas import tpu_sc as plsc

EMBED = 128
NC, NS = 2, 16
NW = NC * NS
K = 512


def _table_body(res_ref, let_ref, row_ref, col_ref, out_ref):
    i = lax.broadcasted_iota(jnp.int32, (480, EMBED), 0)
    s0 = i // 120
    s1 = (i // 30) % 4
    r = (i // 5) % 6
    c = i % 5

    def pick(ref, sel, n):
        acc = jnp.broadcast_to(ref[n - 1, :][None, :], (480, EMBED))
        for k in range(n - 2, -1, -1):
            row = jnp.broadcast_to(ref[k, :][None, :], (480, EMBED))
            acc = jnp.where(sel == k, row, acc)
        return acc

    out_ref[...] = (pick(res_ref, s0, 4) + pick(let_ref, s1, 4)
                    + pick(row_ref, r, 6) + pick(col_ref, c, 5))


def _build_table(result_table, letter_table, row_table, col_table):
    return pl.pallas_call(
        _table_body,
        out_shape=jax.ShapeDtypeStruct((480, EMBED), jnp.float32),
    )(result_table, letter_table, row_table, col_table)


@functools.lru_cache(maxsize=None)
def _make_gather(n_rows):
    assert n_rows % (NW * K) == 0
    rpw = n_rows // NW
    chunks = rpw // K
    mesh = plsc.VectorSubcoreMesh(core_axis_name="c", subcore_axis_name="s",
                                  num_cores=NC, num_subcores=NS)

    @functools.partial(
        pl.kernel,
        out_type=jax.ShapeDtypeStruct((n_rows, EMBED), jnp.float32),
        mesh=mesh,
        scratch_types=[
            pltpu.VMEM((K,), jnp.int32),
            pltpu.VMEM((K,), jnp.int32),
            pltpu.VMEM((K // 128, 128), jnp.int32),
            pltpu.VMEM((K, EMBED), jnp.float32),
            pltpu.SemaphoreType.DMA,
        ],
    )
    def gather(t_hbm, s0_hbm, s1_hbm, out_hbm, s0_v, s1_v, idx_v, rows_v, sem):
        wid = lax.axis_index("s") * NC + lax.axis_index("c")
        w_base = wid * rpw
        lane = lax.iota(jnp.int32, 16)

        def chunk_body(g, carry):
            base = w_base + g * K
            pltpu.sync_copy(s0_hbm.at[pl.ds(base, K)], s0_v)
            pltpu.sync_copy(s1_hbm.at[pl.ds(base, K)], s1_v)
            for j in range(K // 16):
                ii = base + j * 16 + lane
                s0 = s0_v[pl.ds(j * 16, 16)]
                s1 = s1_v[pl.ds(j * 16, 16)]
                fused = (s0 * 4 + s1) * 30 + lax.rem(ii, 30)
                idx_v[j // 8, pl.ds((j % 8) * 16, 16)] = fused
            copies = [
                pltpu.async_copy(t_hbm.at[idx_v.at[j]],
                                 rows_v.at[pl.ds(j * 128, 128)], sem)
                for j in range(K // 128)
            ]
            for cp in copies:
                cp.wait()
            pltpu.sync_copy(rows_v, out_hbm.at[pl.ds(base, K)])
            return carry

        lax.fori_loop(0, chunks, chunk_body, 0)

    return gather


def kernel(state, result_table, letter_table, col_table, row_table):
    bn = state.shape[0]
    table = _build_table(result_table, letter_table, row_table, col_table)
    s0_flat = state[..., 0].reshape(-1)
    s1_flat = state[..., 1].reshape(-1)
    out = _make_gather(bn * 30)(table, s0_flat, s1_flat)
    return out.reshape(bn, 6, 5, EMBED)
